# baseline (speedup 1.0000x reference)
import functools

import jax
import jax.numpy as jnp
from jax.experimental import pallas as pl
from jax.experimental.pallas import tpu as pltpu

NBUF = 4
NCHUNK = 32


def _ring_copy(x_hbm, o_hbm, in_bufs, out_bufs, in_sems, out_sems):
    def start_in(slot, step):
        pltpu.make_async_copy(x_hbm.at[step], in_bufs.at[slot],
                              in_sems.at[slot]).start()

    def wait_in(slot):
        pltpu.make_async_copy(x_hbm.at[0], in_bufs.at[slot],
                              in_sems.at[slot]).wait()

    def start_out(slot, step):
        pltpu.make_async_copy(out_bufs.at[slot], o_hbm.at[step],
                              out_sems.at[slot]).start()

    def wait_out(slot):
        pltpu.make_async_copy(out_bufs.at[slot], o_hbm.at[0],
                              out_sems.at[slot]).wait()

    for i in range(NBUF):
        start_in(i, i)

    def body(step, _):
        slot = jax.lax.rem(step, NBUF)
        wait_in(slot)

        @pl.when(step >= NBUF)
        def _():
            wait_out(slot)

        out_bufs[slot] = in_bufs[slot]
        start_out(slot, step)

        @pl.when(step + NBUF < NCHUNK)
        def _():
            start_in(slot, step + NBUF)
        return ()

    jax.lax.fori_loop(0, NCHUNK, body, ())
    for i in range(NBUF):
        wait_out(jax.lax.rem(NCHUNK - NBUF + i, NBUF))


def kernel(x, w1, b1, w2, b2):
    B, C, H, W = x.shape
    HW = H * W
    rows = B * C // NCHUNK
    x_chunks = x.reshape(NCHUNK, rows, HW)

    out = pl.pallas_call(
        _ring_copy,
        out_shape=jax.ShapeDtypeStruct((NCHUNK, rows, HW), x.dtype),
        in_specs=[pl.BlockSpec(memory_space=pl.ANY)],
        out_specs=pl.BlockSpec(memory_space=pl.ANY),
        scratch_shapes=[
            pltpu.VMEM((NBUF, rows, HW), jnp.float32),
            pltpu.VMEM((NBUF, rows, HW), jnp.float32),
            pltpu.SemaphoreType.DMA((NBUF,)),
            pltpu.SemaphoreType.DMA((NBUF,)),
        ],
        compiler_params=pltpu.CompilerParams(
            vmem_limit_bytes=56 * 1024 * 1024,
        ),
        name="ring_copy_probe",
    )(x_chunks)
    return out.reshape(B, C, H, W)


# speedup vs baseline: 1.7100x; 1.7100x over previous
import functools

import jax
import jax.numpy as jnp
from jax.experimental import pallas as pl
from jax.experimental.pallas import tpu as pltpu


def _se_body(x_ref, w1t_ref, b1_ref, w2t_ref, b2_ref, o_ref, *, inv_hw):
    x = x_ref[...]
    s = jnp.sum(x, axis=-1) * inv_hw
    h = jax.lax.dot_general(s, w1t_ref[...], (((1,), (0,)), ((), ())),
                            preferred_element_type=jnp.float32)
    h = h + b1_ref[...]
    h = h * jax.nn.sigmoid(h)
    z = jax.lax.dot_general(h, w2t_ref[...], (((1,), (0,)), ((), ())),
                            preferred_element_type=jnp.float32)
    e = jax.nn.sigmoid(z + b2_ref[...])
    o_ref[...] = x * e[:, :, None]


def kernel(x, w1, b1, w2, b2):
    B, C, H, W = x.shape
    Cr = w1.shape[0]
    HW = H * W
    inv_hw = 1.0 / float(HW)

    x_flat = x.reshape(B, C, HW)
    w1t = w1.T.astype(jnp.float32)
    w2t = w2.T.astype(jnp.float32)
    b1r = b1.reshape(1, Cr).astype(jnp.float32)
    b2r = b2.reshape(1, C).astype(jnp.float32)

    bt = 4
    while B % bt:
        bt //= 2
    nb = B // bt

    out_flat = pl.pallas_call(
        functools.partial(_se_body, inv_hw=inv_hw),
        out_shape=jax.ShapeDtypeStruct((B, C, HW), x.dtype),
        grid=(nb,),
        in_specs=[
            pl.BlockSpec((bt, C, HW), lambda i: (i, 0, 0)),
            pl.BlockSpec((C, Cr), lambda i: (0, 0)),
            pl.BlockSpec((1, Cr), lambda i: (0, 0)),
            pl.BlockSpec((Cr, C), lambda i: (0, 0)),
            pl.BlockSpec((1, C), lambda i: (0, 0)),
        ],
        out_specs=pl.BlockSpec((bt, C, HW), lambda i: (i, 0, 0)),
        compiler_params=pltpu.CompilerParams(
            dimension_semantics=("arbitrary",),
            vmem_limit_bytes=56 * 1024 * 1024,
        ),
        name="se_fused",
    )(x_flat, w1t, b1r, w2t, b2r)

    return out_flat.reshape(B, C, H, W)
